# baseline (speedup 1.0000x reference)
import functools

import jax
import jax.numpy as jnp
from jax import lax
from jax.experimental import pallas as pl
from jax.experimental.pallas import tpu as pltpu
from jax.experimental.pallas import tpu_sc as plsc

H = 2048
W = 2048
N = 4194304
HW = H * W
HW1 = HW - 1

NW = 32
P = N // NW
CHUNK = 2048
NCHUNK = P // CHUNK
GROUPS = CHUNK // 16
JR = CHUNK // 128

_mesh = plsc.VectorSubcoreMesh(core_axis_name="c", subcore_axis_name="s")
_params = pltpu.CompilerParams(
    needs_layout_passes=False, use_tc_tiling_on_sc=False)

_f32 = jnp.float32
_i32 = jnp.int32


def _scratch():
    per_parity = [
        pltpu.VMEM((2 * CHUNK,), _f32),
        pltpu.VMEM((CHUNK,), _f32),
        pltpu.VMEM((CHUNK,), _f32),
        pltpu.VMEM((CHUNK,), _i32),
        pltpu.VMEM((CHUNK,), _i32),
        pltpu.VMEM((CHUNK,), _i32),
        pltpu.VMEM((CHUNK,), _i32),
        pltpu.VMEM((CHUNK,), _f32),
        pltpu.VMEM((CHUNK,), _f32),
        pltpu.VMEM((CHUNK,), _f32),
        pltpu.VMEM((CHUNK,), _f32),
        pltpu.VMEM((CHUNK,), _f32),
        pltpu.SemaphoreType.DMA,
        pltpu.SemaphoreType.DMA,
        pltpu.SemaphoreType.DMA,
    ]
    return per_parity + per_parity


@functools.partial(
    pl.kernel,
    mesh=_mesh,
    compiler_params=_params,
    out_type=jax.ShapeDtypeStruct((N,), _f32),
    scratch_types=_scratch(),
)
def _sample(x_hbm, img_hbm, out_hbm, *bufs):
    bufA = bufs[:15]
    bufB = bufs[15:]
    wid = lax.axis_index("s") * 2 + lax.axis_index("c")
    wbase = wid * P
    iota = lax.iota(_i32, 16)
    ev = iota * 2

    def fire_x(k, b):
        src = x_hbm.at[pl.ds((wbase + k * CHUNK) * 2, 2 * CHUNK)]
        pltpu.make_async_copy(src, b[0], b[12]).start()

    def wait_x(b):
        src = x_hbm.at[pl.ds(wbase * 2, 2 * CHUNK)]
        pltpu.make_async_copy(src, b[0], b[12]).wait()

    def fire_g(b):
        for t in range(4):
            pltpu.make_async_copy(
                img_hbm.at[b[3 + t]], b[7 + t], b[13]).start()

    def wait_g(b):
        for t in range(4):
            pltpu.make_async_copy(
                img_hbm.at[b[3 + t]], b[7 + t], b[13]).wait()

    def fire_o(k, b):
        dst = out_hbm.at[pl.ds(wbase + k * CHUNK, CHUNK)]
        pltpu.make_async_copy(b[11], dst, b[14]).start()

    def wait_o(b):
        dst = out_hbm.at[pl.ds(wbase, CHUNK)]
        pltpu.make_async_copy(b[11], dst, b[14]).wait()

    def compute(b):
        xv = b[0]

        def grp(g, carry):
            off = g * 32
            gx = plsc.load_gather(xv, [ev + off])
            gy = plsc.load_gather(xv, [ev + (off + 1)])
            ix = ((gx + 1.0) * 0.5) * (W - 1.0)
            iy = ((gy + 1.0) * 0.5) * (H - 1.0)
            xi = ix.astype(_i32)
            yi = iy.astype(_i32)
            s = pl.ds(g * 16, 16)
            b[1][s] = ix - xi.astype(_f32)
            b[2][s] = iy - yi.astype(_f32)
            base = jnp.maximum((yi << 11) + xi, 0)
            b[3][s] = jnp.minimum(base, HW1)
            b[4][s] = jnp.minimum(base + 1, HW1)
            b[5][s] = jnp.minimum(base + W, HW1)
            b[6][s] = jnp.minimum(base + (W + 1), HW1)
            return carry

        lax.fori_loop(0, GROUPS, grp, 0)

    def combine(b):
        def grp(g, carry):
            s = pl.ds(g * 16, 16)
            q00 = b[7][s]
            q01 = b[8][s]
            q10 = b[9][s]
            q11 = b[10][s]
            fx = b[1][s]
            fy = b[2][s]
            t0 = q00 + fx * (q01 - q00)
            t1 = q10 + fx * (q11 - q10)
            b[11][s] = t0 + fy * (t1 - t0)
            return carry

        lax.fori_loop(0, GROUPS, grp, 0)

    def front(k, b, fire_next_x):
        wait_x(b)
        compute(b)
        fire_g(b)
        if fire_next_x:
            fire_x(k + 2, b)

    def drain(k, b, guard):
        wait_g(b)
        if guard is None:
            wait_o(b)
        else:
            pl.when(guard)(lambda: wait_o(b))
        combine(b)
        fire_o(k, b)

    fire_x(0, bufA)
    fire_x(1, bufB)
    front(0, bufA, True)

    def body(c2, carry):
        o = 2 * c2 + 1
        front(o, bufB, True)
        drain(o - 1, bufA, c2 > 0)
        front(o + 1, bufA, True)
        drain(o, bufB, c2 > 0)
        return carry

    lax.fori_loop(0, NCHUNK // 2 - 2, body, 0)
    o = NCHUNK - 3
    front(o, bufB, True)
    drain(o - 1, bufA, None)
    front(o + 1, bufA, False)
    drain(o, bufB, None)
    front(o + 2, bufB, False)
    drain(o + 1, bufA, None)
    drain(o + 2, bufB, None)
    wait_o(bufA)
    wait_o(bufB)


def kernel(x, image):
    return _sample(x.reshape(-1), image.reshape(-1))


# speedup vs baseline: 8.6784x; 8.6784x over previous
import functools

import jax
import jax.numpy as jnp
from jax import lax
from jax.experimental import pallas as pl
from jax.experimental.pallas import tpu as pltpu
from jax.experimental.pallas import tpu_sc as plsc

H = 2048
W = 2048
N = 4194304
HW = H * W
PAD = 2056

NW = 32
P = N // NW
CHUNK = 2048
NCHUNK = P // CHUNK
GROUPS = CHUNK // 16
U = 4

_mesh = plsc.VectorSubcoreMesh(core_axis_name="c", subcore_axis_name="s")
_params = pltpu.CompilerParams(
    needs_layout_passes=False, use_tc_tiling_on_sc=False)

_f32 = jnp.float32
_i32 = jnp.int32


def _scratch():
    per_parity = [
        pltpu.VMEM((CHUNK,), _f32),
        pltpu.VMEM((CHUNK,), _f32),
        pltpu.VMEM((CHUNK,), _f32),
        pltpu.VMEM((CHUNK,), _f32),
        pltpu.VMEM((CHUNK,), _i32),
        pltpu.VMEM((CHUNK,), _i32),
        pltpu.VMEM((CHUNK,), _f32),
        pltpu.VMEM((CHUNK,), _f32),
        pltpu.VMEM((CHUNK,), _f32),
        pltpu.VMEM((CHUNK,), _f32),
        pltpu.VMEM((CHUNK,), _f32),
        pltpu.SemaphoreType.DMA,
        pltpu.SemaphoreType.DMA,
        pltpu.SemaphoreType.DMA,
    ]
    return per_parity + per_parity


@functools.partial(
    pl.kernel,
    mesh=_mesh,
    compiler_params=_params,
    out_type=jax.ShapeDtypeStruct((N,), _f32),
    scratch_types=_scratch(),
)
def _sample(gx_hbm, gy_hbm, img_hbm, out_hbm, *bufs):
    bufA = bufs[:14]
    bufB = bufs[14:]
    wid = lax.axis_index("s") * 2 + lax.axis_index("c")
    wbase = wid * P

    def fire_x(k, b):
        src = pl.ds(wbase + k * CHUNK, CHUNK)
        pltpu.make_async_copy(gx_hbm.at[src], b[0], b[11]).start()
        pltpu.make_async_copy(gy_hbm.at[src], b[1], b[11]).start()

    def wait_x(b):
        src = pl.ds(wbase, CHUNK)
        pltpu.make_async_copy(gx_hbm.at[src], b[0], b[11]).wait()
        pltpu.make_async_copy(gy_hbm.at[src], b[1], b[11]).wait()

    _taps = ((0, 0), (0, 1), (W, 0), (W, 1))

    def fire_g(b):
        for t, (off, i) in enumerate(_taps):
            src = img_hbm.at[pl.ds(off, HW + 8)].at[b[4 + i]]
            pltpu.make_async_copy(src, b[6 + t], b[12]).start()

    def wait_g(b):
        for t, (off, i) in enumerate(_taps):
            src = img_hbm.at[pl.ds(off, HW + 8)].at[b[4 + i]]
            pltpu.make_async_copy(src, b[6 + t], b[12]).wait()

    def fire_o(k, b):
        dst = out_hbm.at[pl.ds(wbase + k * CHUNK, CHUNK)]
        pltpu.make_async_copy(b[10], dst, b[13]).start()

    def wait_o(b):
        dst = out_hbm.at[pl.ds(wbase, CHUNK)]
        pltpu.make_async_copy(b[10], dst, b[13]).wait()

    def compute(b):
        def grp(g, carry):
            for u in range(U):
                s = pl.ds((g * U + u) * 16, 16)
                gx = b[0][s]
                gy = b[1][s]
                ix = ((gx + 1.0) * 0.5) * (W - 1.0)
                iy = ((gy + 1.0) * 0.5) * (H - 1.0)
                xi = ix.astype(_i32)
                yi = iy.astype(_i32)
                b[2][s] = ix - xi.astype(_f32)
                b[3][s] = iy - yi.astype(_f32)
                tl = (yi << 11) + xi
                b[4][s] = tl
                b[5][s] = tl + 1
            return carry

        lax.fori_loop(0, GROUPS // U, grp, 0)

    def combine(b):
        def grp(g, carry):
            for u in range(U):
                s = pl.ds((g * U + u) * 16, 16)
                q00 = b[6][s]
                q01 = b[7][s]
                q10 = b[8][s]
                q11 = b[9][s]
                fx = b[2][s]
                fy = b[3][s]
                t0 = q00 + fx * (q01 - q00)
                t1 = q10 + fx * (q11 - q10)
                b[10][s] = t0 + fy * (t1 - t0)
            return carry

        lax.fori_loop(0, GROUPS // U, grp, 0)

    def front(k, b, fire_next_x):
        wait_x(b)
        compute(b)
        fire_g(b)
        if fire_next_x:
            fire_x(k + 2, b)

    def drain(k, b, guard):
        wait_g(b)
        if guard is None:
            wait_o(b)
        else:
            pl.when(guard)(lambda: wait_o(b))
        combine(b)
        fire_o(k, b)

    fire_x(0, bufA)
    fire_x(1, bufB)
    front(0, bufA, True)

    def body(c2, carry):
        o = 2 * c2 + 1
        front(o, bufB, True)
        drain(o - 1, bufA, c2 > 0)
        front(o + 1, bufA, True)
        drain(o, bufB, c2 > 0)
        return carry

    lax.fori_loop(0, NCHUNK // 2 - 2, body, 0)
    o = NCHUNK - 3
    front(o, bufB, True)
    drain(o - 1, bufA, None)
    front(o + 1, bufA, False)
    drain(o, bufB, None)
    front(o + 2, bufB, False)
    drain(o + 1, bufA, None)
    drain(o + 2, bufB, None)
    wait_o(bufA)
    wait_o(bufB)


def kernel(x, image):
    gx = x[:, 0].reshape(-1)
    gy = x[:, 1].reshape(-1)
    img_pad = jnp.concatenate(
        [image.reshape(-1), jnp.zeros((PAD,), _f32)])
    return _sample(gx, gy, img_pad)
